# Initial kernel scaffold; baseline (speedup 1.0000x reference)
import jax
import jax.numpy as jnp
from jax.experimental import pallas as pl


def kernel(z, edge_label_index):
    raise NotImplementedError("write your pallas kernel here")


# speedup vs baseline: 3.4815x; 3.4815x over previous
import functools

import jax
import jax.numpy as jnp
from jax import lax
from jax.experimental import pallas as pl
from jax.experimental.pallas import tpu as pltpu
from jax.experimental.pallas import tpu_sc as plsc

NC = 2
NS = 16
L = 16
NW = NC * NS

C = 80
TP = 17


def _dot_decode(z_hbm, h_hbm, t_hbm, out_hbm, idxh, idxt, bufh, buft, tpose,
                outv, sem, *, per_w, chunks, d_model):
    cid = lax.axis_index("c")
    sid = lax.axis_index("s")
    wid = sid * NC + cid
    base = wid * per_w

    lanes = lax.iota(jnp.int32, L)
    nk = d_model // L

    def chunk_body(j, carry):
        off = base + j * C
        pltpu.sync_copy(h_hbm.at[pl.ds(off, C)], idxh)
        pltpu.sync_copy(t_hbm.at[pl.ds(off, C)], idxt)
        cpy_h = pltpu.async_copy(z_hbm.at[idxh], bufh, sem)
        cpy_t = pltpu.async_copy(z_hbm.at[idxt], buft, sem)
        cpy_h.wait()
        cpy_t.wait()

        def group_body(g, carry2):
            for e_l in range(L):
                e = g * L + e_l
                r = bufh[e, pl.ds(0, L)] * buft[e, pl.ds(0, L)]
                for k in range(1, nk):
                    r = r + bufh[e, pl.ds(k * L, L)] * buft[e, pl.ds(k * L, L)]
                plsc.store_scatter(tpose, [lanes * TP + e_l], r)
            acc = tpose[pl.ds(0, L)]
            for l in range(1, L):
                acc = acc + tpose[pl.ds(l * TP, L)]
            outv[pl.ds(g * L, L)] = acc
            return carry2

        lax.fori_loop(0, C // L, group_body, 0)
        pltpu.sync_copy(outv, out_hbm.at[pl.ds(off, C)])
        return carry

    lax.fori_loop(0, chunks, chunk_body, 0)


def kernel(z, edge_label_index):
    n, d_model = z.shape
    e = edge_label_index.shape[1]
    assert e % (NW * C) == 0 and d_model % L == 0, (e, d_model)
    per_w = e // NW
    chunks = per_w // C

    head = edge_label_index[0]
    tail = edge_label_index[1]

    run = pl.kernel(
        functools.partial(_dot_decode, per_w=per_w, chunks=chunks,
                          d_model=d_model),
        out_type=jax.ShapeDtypeStruct((e,), jnp.float32),
        mesh=plsc.VectorSubcoreMesh(core_axis_name="c", subcore_axis_name="s"),
        compiler_params=pltpu.CompilerParams(needs_layout_passes=False),
        scratch_types=[
            pltpu.VMEM((C,), jnp.int32),
            pltpu.VMEM((C,), jnp.int32),
            pltpu.VMEM((C, d_model), jnp.float32),
            pltpu.VMEM((C, d_model), jnp.float32),
            pltpu.VMEM((L * TP,), jnp.float32),
            pltpu.VMEM((C,), jnp.float32),
            pltpu.SemaphoreType.DMA,
        ],
    )
    return run(z, head, tail)


# speedup vs baseline: 7.8581x; 2.2571x over previous
import functools

import jax
import jax.numpy as jnp
from jax import lax
from jax.experimental import pallas as pl
from jax.experimental.pallas import tpu as pltpu
from jax.experimental.pallas import tpu_sc as plsc

NC = 2
NS = 16
L = 16
NW = NC * NS

C = 80
NBUF = 5
TP = 17


def _dot_decode(z_hbm, h_hbm, t_hbm, out_hbm, idxh, idxt, bufh, buft, tpose,
                outv, sems, *, per_w, chunks, d_model):
    cid = lax.axis_index("c")
    sid = lax.axis_index("s")
    wid = sid * NC + cid
    base = wid * per_w

    lanes = lax.iota(jnp.int32, L)
    nk = d_model // L

    pltpu.sync_copy(h_hbm.at[pl.ds(base, per_w)], idxh)
    pltpu.sync_copy(t_hbm.at[pl.ds(base, per_w)], idxt)

    def fire(j, b):
        pltpu.async_copy(z_hbm.at[idxh.at[pl.ds(j * C, C)]], bufh[b], sems[b])
        pltpu.async_copy(z_hbm.at[idxt.at[pl.ds(j * C, C)]], buft[b], sems[b])

    def drain(b):
        pltpu.make_async_copy(z_hbm.at[pl.ds(0, C)], bufh[b], sems[b]).wait()
        pltpu.make_async_copy(z_hbm.at[pl.ds(0, C)], buft[b], sems[b]).wait()

    def compute(j, b):
        def group_body(g, carry2):
            for e_l in range(L):
                e = g * L + e_l
                r = bufh[b][e, pl.ds(0, L)] * buft[b][e, pl.ds(0, L)]
                for k in range(1, nk):
                    r = r + (bufh[b][e, pl.ds(k * L, L)]
                             * buft[b][e, pl.ds(k * L, L)])
                plsc.store_scatter(tpose, [lanes * TP + e_l], r)
            acc = tpose[pl.ds(0, L)]
            for l in range(1, L):
                acc = acc + tpose[pl.ds(l * TP, L)]
            outv[pl.ds(g * L, L)] = acc
            return carry2

        lax.fori_loop(0, C // L, group_body, 0)
        pltpu.sync_copy(outv, out_hbm.at[pl.ds(base + j * C, C)])

    for b in range(NBUF):
        fire(b, b)

    def ring_body(jq, carry):
        j0 = jq * NBUF
        for b in range(NBUF):
            drain(b)
            compute(j0 + b, b)
            fire(j0 + b + NBUF, b)
        return carry

    lax.fori_loop(0, chunks // NBUF - 1, ring_body, 0)

    j0 = chunks - NBUF
    for b in range(NBUF):
        drain(b)
        compute(j0 + b, b)


def kernel(z, edge_label_index):
    n, d_model = z.shape
    e = edge_label_index.shape[1]
    assert e % (NW * C * NBUF) == 0 and d_model % L == 0, (e, d_model)
    per_w = e // NW
    chunks = per_w // C

    head = edge_label_index[0]
    tail = edge_label_index[1]

    run = pl.kernel(
        functools.partial(_dot_decode, per_w=per_w, chunks=chunks,
                          d_model=d_model),
        out_type=jax.ShapeDtypeStruct((e,), jnp.float32),
        mesh=plsc.VectorSubcoreMesh(core_axis_name="c", subcore_axis_name="s"),
        compiler_params=pltpu.CompilerParams(needs_layout_passes=False),
        scratch_types=[
            pltpu.VMEM((per_w,), jnp.int32),
            pltpu.VMEM((per_w,), jnp.int32),
            [pltpu.VMEM((C, d_model), jnp.float32) for _ in range(NBUF)],
            [pltpu.VMEM((C, d_model), jnp.float32) for _ in range(NBUF)],
            pltpu.VMEM((L * TP,), jnp.float32),
            pltpu.VMEM((C,), jnp.float32),
            [pltpu.SemaphoreType.DMA for _ in range(NBUF)],
        ],
    )
    return run(z, head, tail)
